# Initial kernel scaffold; baseline (speedup 1.0000x reference)
import jax
import jax.numpy as jnp
from jax.experimental import pallas as pl


def kernel(f_atoms, f_bonds, a2b, b2a, b2revb, W_in_w, W_in_b, W_msg_w, W_msg_b, W_atom_w, W_atom_b, node_w, node_b, edge_w, edge_b):
    raise NotImplementedError("write your pallas kernel here")


# speedup vs baseline: 1.2646x; 1.2646x over previous
import functools

import jax
import jax.numpy as jnp
from jax import lax
from jax.experimental import pallas as pl
from jax.experimental.pallas import tpu as pltpu
from jax.experimental.pallas import tpu_sc as plsc

N_ATOMS = 10000
N_EDGES = 320000
HIDDEN = 128
STEPS = 3

NC, NS = 2, 16
NW = NC * NS
EPW = N_EDGES // NW
CH = 80
NCHUNK = EPW // CH
APT = 624
ATL = N_ATOMS - NS * APT

_mesh = plsc.VectorSubcoreMesh(core_axis_name="c", subcore_axis_name="s")


@functools.partial(
    pl.kernel,
    out_type=(
        jax.ShapeDtypeStruct((NC * N_ATOMS, HIDDEN), jnp.float32),
        jax.ShapeDtypeStruct((N_EDGES, HIDDEN), jnp.float32),
    ),
    mesh=_mesh,
    scratch_types=[
        pltpu.VMEM((CH, HIDDEN), jnp.float32),
        pltpu.VMEM((CH,), jnp.int32),
        pltpu.VMEM((CH, HIDDEN), jnp.float32),
        pltpu.VMEM((CH,), jnp.int32),
        pltpu.VMEM_SHARED((N_ATOMS, HIDDEN), jnp.float32),
        pltpu.SemaphoreType.DMA,
    ],
)
def _sc_scatter_rev(h_hbm, b2a_hbm, b2revb_hbm, zeros_hbm,
                    part_hbm, r_hbm,
                    hv, div, rv, riv, acc, sem):
    cid = lax.axis_index("c")
    sid = lax.axis_index("s")
    wid = cid * NS + sid
    ebase = wid * EPW
    pltpu.sync_copy(zeros_hbm.at[pl.ds(sid * APT, APT)],
                    acc.at[pl.ds(sid * APT, APT)])

    @pl.when(sid == NS - 1)
    def _():
        pltpu.sync_copy(zeros_hbm.at[pl.ds(NS * APT, ATL)],
                        acc.at[pl.ds(NS * APT, ATL)])
    plsc.subcore_barrier()

    def body(i, carry):
        off = ebase + i * CH
        pltpu.sync_copy(b2revb_hbm.at[pl.ds(off, CH)], riv)
        pltpu.async_copy(b2a_hbm.at[riv], div, sem).wait()
        pltpu.sync_copy(h_hbm.at[pl.ds(off, CH)], hv)
        pltpu.sync_copy(hv, acc.at[div], add=True)
        pltpu.async_copy(h_hbm.at[riv], rv, sem).wait()
        pltpu.sync_copy(rv, r_hbm.at[pl.ds(off, CH)])
        return carry

    lax.fori_loop(0, NCHUNK, body, 0)
    plsc.subcore_barrier()
    pltpu.sync_copy(acc.at[pl.ds(sid * APT, APT)],
                    part_hbm.at[pl.ds(cid * N_ATOMS + sid * APT, APT)])

    @pl.when(sid == NS - 1)
    def _():
        pltpu.sync_copy(acc.at[pl.ds(NS * APT, ATL)],
                        part_hbm.at[pl.ds(cid * N_ATOMS + NS * APT, ATL)])


@functools.partial(
    pl.kernel,
    out_type=jax.ShapeDtypeStruct((NC * N_ATOMS, HIDDEN), jnp.float32),
    mesh=_mesh,
    scratch_types=[
        pltpu.VMEM((CH, HIDDEN), jnp.float32),
        pltpu.VMEM((CH,), jnp.int32),
        pltpu.VMEM((CH,), jnp.int32),
        pltpu.VMEM_SHARED((N_ATOMS, HIDDEN), jnp.float32),
        pltpu.SemaphoreType.DMA,
    ],
)
def _sc_scatter_only(h_hbm, b2a_hbm, b2revb_hbm, zeros_hbm, part_hbm,
                     hv, div, riv, acc, sem):
    cid = lax.axis_index("c")
    sid = lax.axis_index("s")
    wid = cid * NS + sid
    ebase = wid * EPW
    pltpu.sync_copy(zeros_hbm.at[pl.ds(sid * APT, APT)],
                    acc.at[pl.ds(sid * APT, APT)])

    @pl.when(sid == NS - 1)
    def _():
        pltpu.sync_copy(zeros_hbm.at[pl.ds(NS * APT, ATL)],
                        acc.at[pl.ds(NS * APT, ATL)])
    plsc.subcore_barrier()

    def body(i, carry):
        off = ebase + i * CH
        pltpu.sync_copy(b2revb_hbm.at[pl.ds(off, CH)], riv)
        pltpu.async_copy(b2a_hbm.at[riv], div, sem).wait()
        pltpu.sync_copy(h_hbm.at[pl.ds(off, CH)], hv)
        pltpu.sync_copy(hv, acc.at[div], add=True)
        return carry

    lax.fori_loop(0, NCHUNK, body, 0)
    plsc.subcore_barrier()
    pltpu.sync_copy(acc.at[pl.ds(sid * APT, APT)],
                    part_hbm.at[pl.ds(cid * N_ATOMS + sid * APT, APT)])

    @pl.when(sid == NS - 1)
    def _():
        pltpu.sync_copy(acc.at[pl.ds(NS * APT, ATL)],
                        part_hbm.at[pl.ds(cid * N_ATOMS + NS * APT, ATL)])


@functools.partial(
    pl.kernel,
    out_type=jax.ShapeDtypeStruct((N_EDGES, HIDDEN), jnp.float32),
    mesh=_mesh,
    scratch_types=[
        pltpu.VMEM((CH, HIDDEN), jnp.float32),
        pltpu.VMEM((CH,), jnp.int32),
        pltpu.SemaphoreType.DMA,
    ],
)
def _sc_gather(dms_hbm, b2a_hbm, g_hbm, gv, giv, sem):
    cid = lax.axis_index("c")
    sid = lax.axis_index("s")
    wid = cid * NS + sid
    ebase = wid * EPW

    def body(i, carry):
        off = ebase + i * CH
        pltpu.sync_copy(b2a_hbm.at[pl.ds(off, CH)], giv)
        pltpu.async_copy(dms_hbm.at[giv], gv, sem).wait()
        pltpu.sync_copy(gv, g_hbm.at[pl.ds(off, CH)])
        return carry

    lax.fori_loop(0, NCHUNK, body, 0)


BR = 1000
BA = 1000


def _tc_in_body(fb, w, b, o):
    o[...] = jnp.maximum(
        jnp.dot(fb[...], w[...], preferred_element_type=jnp.float32) + b[...], 0.0)


def _tc_upd_body(h, g, r, w, b, o):
    o[...] = jnp.maximum(
        h[...] + jnp.dot(g[...] - r[...], w[...],
                         preferred_element_type=jnp.float32) + b[...], 0.0)


def _tc_add_body(pa, pb, o):
    o[...] = pa[...] + pb[...]


def _tc_atom_body(pa, pb, fa, wt, wb, bb, nw, nb, ha, npred):
    h_atom = jnp.maximum(
        jnp.dot(pa[...] + pb[...], wt[...], preferred_element_type=jnp.float32)
        + jnp.dot(fa[...], wb[...], preferred_element_type=jnp.float32)
        + bb[...], 0.0)
    ha[...] = h_atom
    npred[...] = jnp.dot(h_atom, nw[...], preferred_element_type=jnp.float32) + nb[...]


def _tc_edge_body(h, w, b, o):
    o[...] = jnp.dot(h[...], w[...], preferred_element_type=jnp.float32) + b[...]


def _full(shape):
    return pl.BlockSpec(shape, lambda i: (0, 0))


def kernel(f_atoms, f_bonds, a2b, b2a, b2revb,
           W_in_w, W_in_b, W_msg_w, W_msg_b,
           W_atom_w, W_atom_b, node_w, node_b, edge_w, edge_b):
    del a2b
    FB = f_bonds.shape[1]
    zeros_a = jnp.zeros((N_ATOMS, HIDDEN), jnp.float32)
    b2a = b2a.astype(jnp.int32)
    b2revb = b2revb.astype(jnp.int32)

    h = pl.pallas_call(
        _tc_in_body,
        grid=(N_EDGES // BR,),
        in_specs=[pl.BlockSpec((BR, FB), lambda i: (i, 0)),
                  _full((FB, HIDDEN)), _full((1, HIDDEN))],
        out_specs=pl.BlockSpec((BR, HIDDEN), lambda i: (i, 0)),
        out_shape=jax.ShapeDtypeStruct((N_EDGES, HIDDEN), jnp.float32),
    )(f_bonds, W_in_w, W_in_b.reshape(1, HIDDEN))

    add_call = pl.pallas_call(
        _tc_add_body,
        grid=(N_ATOMS // BA,),
        in_specs=[pl.BlockSpec((BA, HIDDEN), lambda i: (i, 0)),
                  pl.BlockSpec((BA, HIDDEN), lambda i: (i + N_ATOMS // BA, 0))],
        out_specs=pl.BlockSpec((BA, HIDDEN), lambda i: (i, 0)),
        out_shape=jax.ShapeDtypeStruct((N_ATOMS, HIDDEN), jnp.float32),
    )

    upd_call = pl.pallas_call(
        _tc_upd_body,
        grid=(N_EDGES // BR,),
        in_specs=[pl.BlockSpec((BR, HIDDEN), lambda i: (i, 0)),
                  pl.BlockSpec((BR, HIDDEN), lambda i: (i, 0)),
                  pl.BlockSpec((BR, HIDDEN), lambda i: (i, 0)),
                  _full((HIDDEN, HIDDEN)), _full((1, HIDDEN))],
        out_specs=pl.BlockSpec((BR, HIDDEN), lambda i: (i, 0)),
        out_shape=jax.ShapeDtypeStruct((N_EDGES, HIDDEN), jnp.float32),
    )

    msg_b = W_msg_b.reshape(1, HIDDEN)
    for _ in range(STEPS):
        part, r = _sc_scatter_rev(h, b2a, b2revb, zeros_a)
        dms = add_call(part, part)
        g = _sc_gather(dms, b2a)
        h = upd_call(h, g, r, W_msg_w, msg_b)

    part = _sc_scatter_only(h, b2a, b2revb, zeros_a)

    h_atom, node_pred = pl.pallas_call(
        _tc_atom_body,
        grid=(N_ATOMS // BA,),
        in_specs=[pl.BlockSpec((BA, HIDDEN), lambda i: (i, 0)),
                  pl.BlockSpec((BA, HIDDEN), lambda i: (i + N_ATOMS // BA, 0)),
                  pl.BlockSpec((BA, f_atoms.shape[1]), lambda i: (i, 0)),
                  _full((HIDDEN, HIDDEN)), _full((f_atoms.shape[1], HIDDEN)),
                  _full((1, HIDDEN)),
                  _full((HIDDEN, node_w.shape[1])), _full((1, node_w.shape[1]))],
        out_specs=[pl.BlockSpec((BA, HIDDEN), lambda i: (i, 0)),
                   pl.BlockSpec((BA, node_w.shape[1]), lambda i: (i, 0))],
        out_shape=[jax.ShapeDtypeStruct((N_ATOMS, HIDDEN), jnp.float32),
                   jax.ShapeDtypeStruct((N_ATOMS, node_w.shape[1]), jnp.float32)],
    )(part, part, f_atoms, W_atom_w[:HIDDEN], W_atom_w[HIDDEN:],
      W_atom_b.reshape(1, HIDDEN), node_w, node_b.reshape(1, -1))

    edge_pred = pl.pallas_call(
        _tc_edge_body,
        grid=(N_EDGES // BR,),
        in_specs=[pl.BlockSpec((BR, HIDDEN), lambda i: (i, 0)),
                  _full((HIDDEN, edge_w.shape[1])), _full((1, edge_w.shape[1]))],
        out_specs=pl.BlockSpec((BR, edge_w.shape[1]), lambda i: (i, 0)),
        out_shape=jax.ShapeDtypeStruct((N_EDGES, edge_w.shape[1]), jnp.float32),
    )(h, edge_w, edge_b.reshape(1, -1))

    return (node_pred, edge_pred, h_atom)


# speedup vs baseline: 1.6134x; 1.2758x over previous
import functools

import jax
import jax.numpy as jnp
from jax import lax
from jax.experimental import pallas as pl
from jax.experimental.pallas import tpu as pltpu
from jax.experimental.pallas import tpu_sc as plsc

N_ATOMS = 10000
N_EDGES = 320000
HIDDEN = 128
STEPS = 3

NC, NS = 2, 16
NW = NC * NS
EPW = N_EDGES // NW
CH = 80
NCHUNK = EPW // CH
APT = 624
ATL = N_ATOMS - NS * APT
HL = HIDDEN // 16

_mesh = plsc.VectorSubcoreMesh(core_axis_name="c", subcore_axis_name="s")


@functools.partial(
    pl.kernel,
    out_type=jax.ShapeDtypeStruct((N_EDGES,), jnp.int32),
    mesh=_mesh,
    scratch_types=[
        pltpu.VMEM((CH,), jnp.int32),
        pltpu.VMEM((CH,), jnp.int32),
        pltpu.SemaphoreType.DMA,
    ],
)
def _sc_didx(b2a_hbm, b2revb_hbm, didx_hbm, riv, div, sem):
    wid = lax.axis_index("c") * NS + lax.axis_index("s")
    ebase = wid * EPW

    def body(i, carry):
        off = ebase + i * CH
        pltpu.sync_copy(b2revb_hbm.at[pl.ds(off, CH)], riv)
        pltpu.async_copy(b2a_hbm.at[riv], div, sem).wait()
        pltpu.sync_copy(div, didx_hbm.at[pl.ds(off, CH)])
        return carry

    lax.fori_loop(0, NCHUNK, body, 0)


@functools.partial(
    pl.kernel,
    out_type=jax.ShapeDtypeStruct((NC * N_ATOMS, HIDDEN), jnp.float32),
    mesh=_mesh,
    scratch_types=[
        pltpu.VMEM((CH, HIDDEN), jnp.float32),
        pltpu.VMEM((CH,), jnp.int32),
        pltpu.VMEM_SHARED((N_ATOMS, HIDDEN), jnp.float32),
        pltpu.SemaphoreType.DMA,
    ],
)
def _sc_scatter(h_hbm, didx_hbm, zeros_hbm, part_hbm, hv, div, acc, sem):
    cid = lax.axis_index("c")
    sid = lax.axis_index("s")
    ebase = (cid * NS + sid) * EPW
    pltpu.sync_copy(zeros_hbm.at[pl.ds(sid * APT, APT)],
                    acc.at[pl.ds(sid * APT, APT)])

    @pl.when(sid == NS - 1)
    def _():
        pltpu.sync_copy(zeros_hbm.at[pl.ds(NS * APT, ATL)],
                        acc.at[pl.ds(NS * APT, ATL)])
    plsc.subcore_barrier()

    def body(i, carry):
        off = ebase + i * CH
        pltpu.sync_copy(didx_hbm.at[pl.ds(off, CH)], div)
        pltpu.sync_copy(h_hbm.at[pl.ds(off, CH)], hv)
        pltpu.sync_copy(hv, acc.at[div], add=True)
        return carry

    lax.fori_loop(0, NCHUNK, body, 0)
    plsc.subcore_barrier()
    pltpu.sync_copy(acc.at[pl.ds(sid * APT, APT)],
                    part_hbm.at[pl.ds(cid * N_ATOMS + sid * APT, APT)])

    @pl.when(sid == NS - 1)
    def _():
        pltpu.sync_copy(acc.at[pl.ds(NS * APT, ATL)],
                        part_hbm.at[pl.ds(cid * N_ATOMS + NS * APT, ATL)])


@functools.partial(
    pl.kernel,
    out_type=jax.ShapeDtypeStruct((N_EDGES, HIDDEN), jnp.float32),
    mesh=_mesh,
    scratch_types=[
        pltpu.VMEM((CH, HIDDEN), jnp.float32),
        pltpu.VMEM((CH, HIDDEN), jnp.float32),
        pltpu.VMEM((CH, HIDDEN), jnp.float32),
        pltpu.VMEM((CH,), jnp.int32),
        pltpu.VMEM((CH,), jnp.int32),
        pltpu.SemaphoreType.DMA,
        pltpu.SemaphoreType.DMA,
    ],
)
def _sc_update(h_hbm, z_hbm, y_hbm, b2a_hbm, b2revb_hbm, out_hbm,
               hv, zv, yv, av, riv, sem_a, sem_b):
    wid = lax.axis_index("c") * NS + lax.axis_index("s")
    ebase = wid * EPW

    def body(i, carry):
        off = ebase + i * CH
        pltpu.sync_copy(b2a_hbm.at[pl.ds(off, CH)], av)
        pltpu.sync_copy(b2revb_hbm.at[pl.ds(off, CH)], riv)
        ga = pltpu.async_copy(z_hbm.at[av], zv, sem_a)
        gb = pltpu.async_copy(y_hbm.at[riv], yv, sem_b)
        pltpu.sync_copy(h_hbm.at[pl.ds(off, CH)], hv)
        ga.wait()
        gb.wait()

        def row(rr, c2):
            for j in range(HL):
                sl = pl.ds(j * 16, 16)
                hv[rr, sl] = jnp.maximum(hv[rr, sl] + zv[rr, sl] - yv[rr, sl],
                                         0.0)
            return c2

        lax.fori_loop(0, CH, row, 0)
        pltpu.sync_copy(hv, out_hbm.at[pl.ds(off, CH)])
        return carry

    lax.fori_loop(0, NCHUNK, body, 0)


BR = 1000
BA = 1000


def _tc_in_body(fb, w, b, o):
    o[...] = jnp.maximum(
        jnp.dot(fb[...], w[...], preferred_element_type=jnp.float32) + b[...], 0.0)


def _tc_mm_body(x, w, o):
    o[...] = jnp.dot(x[...], w[...], preferred_element_type=jnp.float32)


def _tc_z_body(pa, pb, w, b, o):
    o[...] = jnp.dot(pa[...] + pb[...], w[...],
                     preferred_element_type=jnp.float32) + b[...]


def _tc_atom_body(pa, pb, fa, wt, wb, bb, nw, nb, ha, npred):
    h_atom = jnp.maximum(
        jnp.dot(pa[...] + pb[...], wt[...], preferred_element_type=jnp.float32)
        + jnp.dot(fa[...], wb[...], preferred_element_type=jnp.float32)
        + bb[...], 0.0)
    ha[...] = h_atom
    npred[...] = jnp.dot(h_atom, nw[...], preferred_element_type=jnp.float32) + nb[...]


def _tc_edge_body(h, w, b, o):
    o[...] = jnp.dot(h[...], w[...], preferred_element_type=jnp.float32) + b[...]


def _full(shape):
    return pl.BlockSpec(shape, lambda i: (0, 0))


def kernel(f_atoms, f_bonds, a2b, b2a, b2revb,
           W_in_w, W_in_b, W_msg_w, W_msg_b,
           W_atom_w, W_atom_b, node_w, node_b, edge_w, edge_b):
    del a2b
    FB = f_bonds.shape[1]
    zeros_a = jnp.zeros((N_ATOMS, HIDDEN), jnp.float32)
    b2a = b2a.astype(jnp.int32)
    b2revb = b2revb.astype(jnp.int32)

    didx = _sc_didx(b2a, b2revb)

    h = pl.pallas_call(
        _tc_in_body,
        grid=(N_EDGES // BR,),
        in_specs=[pl.BlockSpec((BR, FB), lambda i: (i, 0)),
                  _full((FB, HIDDEN)), _full((1, HIDDEN))],
        out_specs=pl.BlockSpec((BR, HIDDEN), lambda i: (i, 0)),
        out_shape=jax.ShapeDtypeStruct((N_EDGES, HIDDEN), jnp.float32),
    )(f_bonds, W_in_w, W_in_b.reshape(1, HIDDEN))

    mm_call = pl.pallas_call(
        _tc_mm_body,
        grid=(N_EDGES // BR,),
        in_specs=[pl.BlockSpec((BR, HIDDEN), lambda i: (i, 0)),
                  _full((HIDDEN, HIDDEN))],
        out_specs=pl.BlockSpec((BR, HIDDEN), lambda i: (i, 0)),
        out_shape=jax.ShapeDtypeStruct((N_EDGES, HIDDEN), jnp.float32),
    )

    z_call = pl.pallas_call(
        _tc_z_body,
        grid=(N_ATOMS // BA,),
        in_specs=[pl.BlockSpec((BA, HIDDEN), lambda i: (i, 0)),
                  pl.BlockSpec((BA, HIDDEN), lambda i: (i + N_ATOMS // BA, 0)),
                  _full((HIDDEN, HIDDEN)), _full((1, HIDDEN))],
        out_specs=pl.BlockSpec((BA, HIDDEN), lambda i: (i, 0)),
        out_shape=jax.ShapeDtypeStruct((N_ATOMS, HIDDEN), jnp.float32),
    )

    msg_b = W_msg_b.reshape(1, HIDDEN)
    for _ in range(STEPS):
        part = _sc_scatter(h, didx, zeros_a)
        y = mm_call(h, W_msg_w)
        z = z_call(part, part, W_msg_w, msg_b)
        h = _sc_update(h, z, y, b2a, b2revb)

    part = _sc_scatter(h, didx, zeros_a)

    h_atom, node_pred = pl.pallas_call(
        _tc_atom_body,
        grid=(N_ATOMS // BA,),
        in_specs=[pl.BlockSpec((BA, HIDDEN), lambda i: (i, 0)),
                  pl.BlockSpec((BA, HIDDEN), lambda i: (i + N_ATOMS // BA, 0)),
                  pl.BlockSpec((BA, f_atoms.shape[1]), lambda i: (i, 0)),
                  _full((HIDDEN, HIDDEN)), _full((f_atoms.shape[1], HIDDEN)),
                  _full((1, HIDDEN)),
                  _full((HIDDEN, node_w.shape[1])), _full((1, node_w.shape[1]))],
        out_specs=[pl.BlockSpec((BA, HIDDEN), lambda i: (i, 0)),
                   pl.BlockSpec((BA, node_w.shape[1]), lambda i: (i, 0))],
        out_shape=[jax.ShapeDtypeStruct((N_ATOMS, HIDDEN), jnp.float32),
                   jax.ShapeDtypeStruct((N_ATOMS, node_w.shape[1]), jnp.float32)],
    )(part, part, f_atoms, W_atom_w[:HIDDEN], W_atom_w[HIDDEN:],
      W_atom_b.reshape(1, HIDDEN), node_w, node_b.reshape(1, -1))

    edge_pred = pl.pallas_call(
        _tc_edge_body,
        grid=(N_EDGES // BR,),
        in_specs=[pl.BlockSpec((BR, HIDDEN), lambda i: (i, 0)),
                  _full((HIDDEN, edge_w.shape[1])), _full((1, edge_w.shape[1]))],
        out_specs=pl.BlockSpec((BR, edge_w.shape[1]), lambda i: (i, 0)),
        out_shape=jax.ShapeDtypeStruct((N_EDGES, edge_w.shape[1]), jnp.float32),
    )(h, edge_w, edge_b.reshape(1, -1))

    return (node_pred, edge_pred, h_atom)
